# baseline (speedup 1.0000x reference)
import math

import numpy as np

import jax
import jax.numpy as jnp
from jax.experimental import pallas as pl
from jax.experimental.pallas import tpu as pltpu
from jax.sharding import Mesh, NamedSharding, PartitionSpec as P

try:
    from jax.experimental.shard_map import shard_map as _shard_map
except ImportError:
    from jax import shard_map as _shard_map

_TRIGRAMS = ("qian", "kun", "zhen", "gen", "kan", "li", "xun", "dui")

_LANES = 512
_BLOCK_ROWS = 1024
_VMEM_LIMIT_BYTES = 56 * 1024 * 1024
_INV_SQRT2 = 1.0 / math.sqrt(2.0)


def _fused_kernel(x_ref, *o_refs):
    v = x_ref[...].astype(jnp.float32)
    o = dict(zip(_TRIGRAMS, o_refs))

    neg = v < 0.0
    E = jnp.exp(-jnp.abs(v))
    recip = 1.0 / (1.0 + E)

    sig = jnp.where(neg, E * recip, recip)
    o["kun"][...] = sig.astype(o["kun"].dtype)

    A = E * E
    tanh_abs = (1.0 - A) / (1.0 + A)
    tanh_v = jnp.where(neg, -tanh_abs, tanh_abs)
    o["kan"][...] = tanh_v.astype(o["kan"].dtype)

    elu_v = jnp.where(neg, E - 1.0, v)
    o["qian"][...] = (elu_v + 0.5 * tanh_v).astype(o["qian"].dtype)
    o["dui"][...] = (elu_v + 0.2 * v).astype(o["dui"].dtype)

    o["zhen"][...] = jnp.where(neg, 0.2 * v, v).astype(o["zhen"].dtype)

    o["xun"][...] = (jnp.maximum(v, 0.0)
                     + jnp.log(1.0 + E)).astype(o["xun"].dtype)

    E12 = jnp.exp(-1.2 * jnp.abs(v))
    r12 = 1.0 / (1.0 + E12)
    sig12 = jnp.where(neg, E12 * r12, r12)
    o["li"][...] = (v * sig12).astype(o["li"].dtype)

    c = jnp.clip(v, -5.0, 5.0)
    z = jnp.abs(c) * _INV_SQRT2
    t = 1.0 / (1.0 + 0.3275911 * z)
    poly = t * (0.254829592 + t * (-0.284496736 + t * (1.421413741
               + t * (-1.453152027 + t * 1.061405429))))
    erf_abs = 1.0 - poly * jnp.exp(-z * z)
    erf_z = jnp.where(c >= 0.0, erf_abs, -erf_abs)
    o["gen"][...] = (0.5 * c * (1.0 + erf_z)).astype(o["gen"].dtype)


def _call_fused(x2):
    rows = x2.shape[0]
    block_rows = rows if rows <= _BLOCK_ROWS else _BLOCK_ROWS
    grid = (pl.cdiv(rows, block_rows),)
    spec = pl.BlockSpec((block_rows, _LANES), lambda i: (i, 0))
    return pl.pallas_call(
        _fused_kernel,
        out_shape=tuple(
            jax.ShapeDtypeStruct(x2.shape, x2.dtype) for _ in _TRIGRAMS),
        grid=grid,
        in_specs=[spec],
        out_specs=tuple(spec for _ in _TRIGRAMS),
        compiler_params=pltpu.CompilerParams(
            dimension_semantics=("parallel",),
            vmem_limit_bytes=_VMEM_LIMIT_BYTES),
    )(x2)


def kernel(x):
    orig_shape = x.shape
    flat = x.reshape(-1)
    n = flat.size
    rows = pl.cdiv(n, _LANES)
    padded_n = rows * _LANES
    if padded_n != n:
        flat = jnp.pad(flat, (0, padded_n - n))
    x2 = flat.reshape(rows, _LANES)

    devs = jax.devices()
    ndev = 2 if (len(devs) >= 2 and rows % 2 == 0) else 1

    if ndev == 1:
        outs = _call_fused(x2)
    else:
        mesh = Mesh(np.array(devs[:2]), ("d",))
        x2 = jax.lax.with_sharding_constraint(
            x2, NamedSharding(mesh, P("d", None)))
        sharded = _shard_map(
            _call_fused, mesh=mesh,
            in_specs=P("d", None),
            out_specs=tuple(P("d", None) for _ in _TRIGRAMS),
            check_rep=False)
        outs = sharded(x2)

    def _restore(o2):
        if padded_n == n:
            return o2.reshape(orig_shape)
        return o2.reshape(-1)[:n].reshape(orig_shape)

    return {name: _restore(o) for name, o in zip(_TRIGRAMS, outs)}


# speedup vs baseline: 4.6915x; 4.6915x over previous
import math

import jax
import jax.numpy as jnp
from jax.experimental import pallas as pl
from jax.experimental.pallas import tpu as pltpu

_TRIGRAMS = ("qian", "kun", "zhen", "gen", "kan", "li", "xun", "dui")

_LANES = 512
_BLOCK_ROWS = 1024
_VMEM_LIMIT_BYTES = 56 * 1024 * 1024
_INV_SQRT2 = 1.0 / math.sqrt(2.0)


def _fused_kernel(x_ref, *o_refs):
    v = x_ref[...].astype(jnp.float32)
    o = dict(zip(_TRIGRAMS, o_refs))

    neg = v < 0.0
    E = jnp.exp(-jnp.abs(v))
    recip = 1.0 / (1.0 + E)

    sig = jnp.where(neg, E * recip, recip)
    o["kun"][...] = sig.astype(o["kun"].dtype)

    A = E * E
    tanh_abs = (1.0 - A) / (1.0 + A)
    tanh_v = jnp.where(neg, -tanh_abs, tanh_abs)
    o["kan"][...] = tanh_v.astype(o["kan"].dtype)

    elu_v = jnp.where(neg, E - 1.0, v)
    o["qian"][...] = (elu_v + 0.5 * tanh_v).astype(o["qian"].dtype)
    o["dui"][...] = (elu_v + 0.2 * v).astype(o["dui"].dtype)

    o["zhen"][...] = jnp.where(neg, 0.2 * v, v).astype(o["zhen"].dtype)

    o["xun"][...] = (jnp.maximum(v, 0.0)
                     + jnp.log(1.0 + E)).astype(o["xun"].dtype)

    E12 = jnp.exp(-1.2 * jnp.abs(v))
    r12 = 1.0 / (1.0 + E12)
    sig12 = jnp.where(neg, E12 * r12, r12)
    o["li"][...] = (v * sig12).astype(o["li"].dtype)

    c = jnp.clip(v, -5.0, 5.0)
    z = jnp.abs(c) * _INV_SQRT2
    t = 1.0 / (1.0 + 0.3275911 * z)
    poly = t * (0.254829592 + t * (-0.284496736 + t * (1.421413741
               + t * (-1.453152027 + t * 1.061405429))))
    erf_abs = 1.0 - poly * jnp.exp(-z * z)
    erf_z = jnp.where(c >= 0.0, erf_abs, -erf_abs)
    o["gen"][...] = (0.5 * c * (1.0 + erf_z)).astype(o["gen"].dtype)


def _call_fused(x2):
    rows = x2.shape[0]
    block_rows = rows if rows <= _BLOCK_ROWS else _BLOCK_ROWS
    grid = (pl.cdiv(rows, block_rows),)
    spec = pl.BlockSpec((block_rows, _LANES), lambda i: (i, 0))
    return pl.pallas_call(
        _fused_kernel,
        out_shape=tuple(
            jax.ShapeDtypeStruct(x2.shape, x2.dtype) for _ in _TRIGRAMS),
        grid=grid,
        in_specs=[spec],
        out_specs=tuple(spec for _ in _TRIGRAMS),
        compiler_params=pltpu.CompilerParams(
            dimension_semantics=("parallel",),
            vmem_limit_bytes=_VMEM_LIMIT_BYTES),
    )(x2)


def kernel(x):
    orig_shape = x.shape
    flat = x.reshape(-1)
    n = flat.size
    rows = pl.cdiv(n, _LANES)
    padded_n = rows * _LANES
    if padded_n != n:
        flat = jnp.pad(flat, (0, padded_n - n))
    x2 = flat.reshape(rows, _LANES)

    outs = _call_fused(x2)

    def _restore(o2):
        if padded_n == n:
            return o2.reshape(orig_shape)
        return o2.reshape(-1)[:n].reshape(orig_shape)

    return {name: _restore(o) for name, o in zip(_TRIGRAMS, outs)}
